# baseline (speedup 1.0000x reference)
import functools

import jax
import jax.numpy as jnp
from jax import lax
from jax.experimental import pallas as pl
from jax.experimental.pallas import tpu as pltpu
from jax.experimental.pallas import tpu_sc as plsc

BATCH = 16384
INPUT_DIM = 1000000
L = 26
NC = 2
NS = 16
NW = NC * NS
RPT = BATCH // NW
IPT = RPT * L


def _sc_kernel(x_hbm, w_hbm, b_hbm, out_hbm, x_v, vals_v, b_v, out_v, sem):
    wid = lax.axis_index("s") * NC + lax.axis_index("c")

    pltpu.sync_copy(x_hbm.at[wid], x_v)
    pltpu.sync_copy(b_hbm, b_v)

    w_flat = w_hbm.at[0]
    cks = IPT // 4
    copies = [
        pltpu.async_copy(
            w_flat.at[x_v.at[pl.ds(c * cks, cks)]],
            vals_v.at[pl.ds(c * cks, cks)],
            sem,
        )
        for c in range(4)
    ]
    for c in copies:
        c.wait()

    bias = b_v[...]

    def group_body(g, _):
        base = g * 16
        acc = jnp.zeros((16,), jnp.float32)
        for j in range(L):
            acc = acc + vals_v[pl.ds(j * RPT + base, 16)]
        z = (acc + bias) * 2.0
        out_v[pl.ds(base, 16)] = 1.0 / (1.0 + jnp.exp(-z))
        return 0

    lax.fori_loop(0, RPT // 16, group_body, 0)

    pltpu.sync_copy(out_v, out_hbm.at[pl.ds(wid * RPT, RPT)])


@jax.jit
def _run(x3, w_flat, b16):
    mesh = plsc.VectorSubcoreMesh(core_axis_name="c", subcore_axis_name="s")
    f = functools.partial(
        pl.kernel,
        mesh=mesh,
        out_type=jax.ShapeDtypeStruct((BATCH,), jnp.float32),
        scratch_types=[
            pltpu.VMEM((IPT,), jnp.int32),
            pltpu.VMEM((IPT,), jnp.float32),
            pltpu.VMEM((16,), jnp.float32),
            pltpu.VMEM((RPT,), jnp.float32),
            pltpu.SemaphoreType.DMA,
        ],
    )(_sc_kernel)
    return f(x3, w_flat, b16)


def kernel(x, w, b):
    x3 = x.reshape(NW, RPT, L).transpose(0, 2, 1).reshape(NW, IPT)
    w_flat = w.reshape(1, INPUT_DIM)
    b16 = jnp.broadcast_to(b, (16,))
    out = _run(x3, w_flat, b16)
    return out.reshape(BATCH, 1)
